# baseline (speedup 1.0000x reference)
import jax
import jax.numpy as jnp
from jax.experimental import pallas as pl
from jax.experimental.pallas import tpu as pltpu

TOPK = 300
Q = 20000
NCLS = 80
R, C = 32, 625

INT_MIN = -(2**31)
BIG = 2**30


def _topk_kernel(bboxes_ref, logits_ref, bb_out_ref, lg_out_ref, keys_ref):
    logit = logits_ref[0]
    scores = jnp.max(logit.reshape(R, C, NCLS), axis=2)
    ikey = jax.lax.bitcast_convert_type(scores, jnp.int32)
    keys = ikey ^ jax.lax.shift_right_logical(
        jax.lax.shift_right_arithmetic(ikey, 31), 1
    )
    keys_ref[...] = keys
    chunkmax = jnp.max(keys, axis=1, keepdims=True)
    riota = jax.lax.broadcasted_iota(jnp.int32, (R, 1), 0)
    ciota = jax.lax.broadcasted_iota(jnp.int32, (1, C), 1)

    def body(i, chunkmax):
        m = jnp.max(chunkmax)
        r = jnp.min(jnp.where(chunkmax == m, riota, BIG))
        row = keys_ref[pl.ds(r, 1), :]
        c = jnp.min(jnp.where(row == m, ciota, BIG))
        idx = r * C + c
        row = jnp.where(ciota == c, INT_MIN, row)
        keys_ref[pl.ds(r, 1), :] = row
        chunkmax = jnp.where(riota == r, jnp.max(row), chunkmax)
        lg_out_ref[0, pl.ds(i, 1), :] = logits_ref[0, pl.ds(idx, 1), :]
        bb_out_ref[0, pl.ds(i, 1), :] = bboxes_ref[0, pl.ds(idx, 1), :]
        return chunkmax

    jax.lax.fori_loop(0, TOPK, body, chunkmax)


def kernel(bboxes, logits):
    B = bboxes.shape[0]
    bb_out, lg_out = pl.pallas_call(
        _topk_kernel,
        grid=(B,),
        in_specs=[
            pl.BlockSpec((1, Q, 4), lambda b: (b, 0, 0)),
            pl.BlockSpec((1, Q, NCLS), lambda b: (b, 0, 0)),
        ],
        out_specs=[
            pl.BlockSpec((1, TOPK, 4), lambda b: (b, 0, 0)),
            pl.BlockSpec((1, TOPK, NCLS), lambda b: (b, 0, 0)),
        ],
        out_shape=[
            jax.ShapeDtypeStruct((B, TOPK, 4), jnp.float32),
            jax.ShapeDtypeStruct((B, TOPK, NCLS), jnp.float32),
        ],
        scratch_shapes=[pltpu.VMEM((R, C), jnp.int32)],
    )(bboxes, logits)
    return (bb_out, lg_out)


# speedup vs baseline: 1.9507x; 1.9507x over previous
import jax
import jax.numpy as jnp
from jax.experimental import pallas as pl

TOPK = 300
Q = 20000
NCLS = 80
R, C = 40, 500
W = 512

INT_MIN = -(2**31)
BIG = 2**30


def _roll(x, d):
    return jnp.concatenate([x[:, d:], x[:, :d]], axis=1)


def _cmpx(keys, qs, lane, d, wantmax):
    bitd = (lane & d) != 0
    pk = jnp.where(bitd, _roll(keys, W - d), _roll(keys, d))
    pq = jnp.where(bitd, _roll(qs, W - d), _roll(qs, d))
    self_wins = (keys > pk) | ((keys == pk) & (qs < pq))
    take_self = self_wins ^ ~wantmax
    return jnp.where(take_self, keys, pk), jnp.where(take_self, qs, pq)


def _merge_net(keys, qs, lane, asc):
    d = W // 2
    while d >= 1:
        keys, qs = _cmpx(keys, qs, lane, d, ((lane & d) == 0) != asc)
        d //= 2
    return keys, qs


def _asc_flags(nrows, h):
    return jax.lax.broadcasted_iota(jnp.int32, (nrows, 1), 0) >= h


def _topk_kernel(bboxes_ref, logits_ref, bb_out_ref, lg_out_ref):
    logit = logits_ref[0]
    scores = jnp.max(logit.reshape(R, C, NCLS), axis=2)
    ikey = jax.lax.bitcast_convert_type(scores, jnp.int32)
    keys = ikey ^ jax.lax.shift_right_logical(
        jax.lax.shift_right_arithmetic(ikey, 31), 1
    )
    keys = jnp.concatenate(
        [keys, jnp.full((R, W - C), INT_MIN, jnp.int32)], axis=1
    )
    qs = (
        jax.lax.broadcasted_iota(jnp.int32, (R, W), 0) * C
        + jax.lax.broadcasted_iota(jnp.int32, (R, W), 1)
    )
    qs = jnp.where(
        jax.lax.broadcasted_iota(jnp.int32, (R, W), 1) < C, qs, BIG
    )
    lane = jax.lax.broadcasted_iota(jnp.int32, (1, W), 1)

    asc = _asc_flags(R, R // 2)
    k = 2
    while k <= W:
        j = k // 2
        while j >= 1:
            wantmax = (((lane & k) == 0) == ((lane & j) == 0)) != asc
            keys, qs = _cmpx(keys, qs, lane, j, wantmax)
            j //= 2
        k *= 2

    n = R
    while n > 1:
        if n % 2:
            keys = jnp.concatenate(
                [keys, jnp.full((1, W), INT_MIN, jnp.int32)], axis=0
            )
            qs = jnp.concatenate([qs, jnp.full((1, W), BIG, jnp.int32)], axis=0)
            n += 1
        h = n // 2
        ka, qa = keys[:h], qs[:h]
        kb, qb = keys[h:n], qs[h:n]
        a_wins = (ka > kb) | ((ka == kb) & (qa < qb))
        keys = jnp.where(a_wins, ka, kb)
        qs = jnp.where(a_wins, qa, qb)
        n = h
        next_n = n + 1 if (n > 1 and n % 2) else n
        keys, qs = _merge_net(keys, qs, lane, _asc_flags(n, max(next_n // 2, 1)))

    qtop = qs[0:1]

    def body(i, _):
        idx = jnp.min(jnp.where(lane == i, qtop, BIG))
        lg_out_ref[0, pl.ds(i, 1), :] = logits_ref[0, pl.ds(idx, 1), :]
        bb_out_ref[0, pl.ds(i, 1), :] = bboxes_ref[0, pl.ds(idx, 1), :]
        return 0

    jax.lax.fori_loop(0, TOPK, body, 0)


def kernel(bboxes, logits):
    B = bboxes.shape[0]
    bb_out, lg_out = pl.pallas_call(
        _topk_kernel,
        grid=(B,),
        in_specs=[
            pl.BlockSpec((1, Q, 4), lambda b: (b, 0, 0)),
            pl.BlockSpec((1, Q, NCLS), lambda b: (b, 0, 0)),
        ],
        out_specs=[
            pl.BlockSpec((1, TOPK, 4), lambda b: (b, 0, 0)),
            pl.BlockSpec((1, TOPK, NCLS), lambda b: (b, 0, 0)),
        ],
        out_shape=[
            jax.ShapeDtypeStruct((B, TOPK, 4), jnp.float32),
            jax.ShapeDtypeStruct((B, TOPK, NCLS), jnp.float32),
        ],
    )(bboxes, logits)
    return (bb_out, lg_out)
